# Initial kernel scaffold; baseline (speedup 1.0000x reference)
import jax
import jax.numpy as jnp
from jax.experimental import pallas as pl


def kernel(embedding, edge_index, users, pos_items, neg_items):
    raise NotImplementedError("write your pallas kernel here")


# speedup vs baseline: 7.2095x; 7.2095x over previous
import functools

import jax
import jax.numpy as jnp
from jax import lax
from jax.experimental import pallas as pl
from jax.experimental.pallas import tpu as pltpu
from jax.experimental.pallas import tpu_sc as plsc

N_USERS = 25000
N = 50000
E = 800000
D = 64
H = 32
B = 4096
G = 3 * B
CH = 128
NCH = E // CH
NC = 2
NS = 16
CPT = 391
RC = 500
NRC = N // RC
RCPT = 7
GCH = G // CH
F32 = jnp.float32

_mesh = plsc.VectorSubcoreMesh(
    core_axis_name="c", subcore_axis_name="s", num_cores=NC, num_subcores=NS)


def _deg_body(row_e, col_e, zeros16, ones16, deg_out, acc, stg, ones_v, ibuf):
    c = lax.axis_index("c")
    s = lax.axis_index("s")
    pltpu.sync_copy(ones16, ones_v)

    def zero(j, carry):
        k = s + NS * j

        @pl.when(k < NRC)
        def _():
            pltpu.sync_copy(zeros16.at[pl.ds(k * RC, RC)], stg)
            pltpu.sync_copy(stg, acc.at[pl.ds(k * RC, RC)])

        return carry

    lax.fori_loop(0, RCPT, zero, 0)
    plsc.subcore_barrier()

    def work(e_ref):
        def step(j, carry):
            ch = s * CPT + j

            @pl.when(ch < NCH)
            def _():
                pltpu.sync_copy(e_ref.at[pl.ds(ch * CH, CH)], ibuf)
                pltpu.sync_copy(ones_v, acc.at[ibuf], add=True)

            return carry

        lax.fori_loop(0, CPT, step, 0)

    @pl.when(c == 0)
    def _():
        work(row_e)

    @pl.when(c == 1)
    def _():
        work(col_e)

    plsc.subcore_barrier()

    def copy_out(ci):
        def cstep(j, carry):
            k = s + NS * j

            @pl.when(k < NRC)
            def _():
                pltpu.sync_copy(acc.at[pl.ds(k * RC, RC)], stg)
                pltpu.sync_copy(stg, deg_out.at[ci, pl.ds(k * RC, RC)])

            return carry

        lax.fori_loop(0, RCPT, cstep, 0)

    @pl.when(c == 0)
    def _():
        copy_out(0)

    @pl.when(c == 1)
    def _():
        copy_out(1)


_deg_kernel = functools.partial(
    pl.kernel,
    out_type=jax.ShapeDtypeStruct((NC, N, 16), F32),
    mesh=_mesh,
    compiler_params=pltpu.CompilerParams(use_tc_tiling_on_sc=False),
    scratch_types=[
        pltpu.VMEM_SHARED((N, 16), F32),
        pltpu.VMEM((RC, 16), F32),
        pltpu.VMEM((CH, 16), F32),
        pltpu.VMEM((CH,), jnp.int32),
    ],
)(_deg_body)


def _spmm_body(row_e, col_e, u_a, u_b, zeros32, s_out,
               acc, stg, gbuf, cbuf, rbuf):
    c = lax.axis_index("c")
    s = lax.axis_index("s")

    def zero(j, carry):
        k = s + NS * j

        @pl.when(k < NRC)
        def _():
            pltpu.sync_copy(zeros32.at[pl.ds(k * RC, RC)], stg)
            pltpu.sync_copy(stg, acc.at[pl.ds(k * RC, RC)])

        return carry

    lax.fori_loop(0, RCPT, zero, 0)
    plsc.subcore_barrier()

    def work(u_ref):
        def step(j, carry):
            ch = s * CPT + j

            @pl.when(ch < NCH)
            def _():
                pltpu.sync_copy(col_e.at[pl.ds(ch * CH, CH)], cbuf)
                pltpu.sync_copy(row_e.at[pl.ds(ch * CH, CH)], rbuf)
                pltpu.sync_copy(u_ref.at[cbuf], gbuf)
                pltpu.sync_copy(gbuf, acc.at[rbuf], add=True)

            return carry

        lax.fori_loop(0, CPT, step, 0)

    @pl.when(c == 0)
    def _():
        work(u_a)

    @pl.when(c == 1)
    def _():
        work(u_b)

    plsc.subcore_barrier()

    def copy_out(ci):
        def cstep(j, carry):
            k = s + NS * j

            @pl.when(k < NRC)
            def _():
                pltpu.sync_copy(acc.at[pl.ds(k * RC, RC)], stg)
                pltpu.sync_copy(stg, s_out.at[ci, pl.ds(k * RC, RC)])

            return carry

        lax.fori_loop(0, RCPT, cstep, 0)

    @pl.when(c == 0)
    def _():
        copy_out(0)

    @pl.when(c == 1)
    def _():
        copy_out(1)


_spmm_kernel = functools.partial(
    pl.kernel,
    out_type=jax.ShapeDtypeStruct((NC, N, H), F32),
    mesh=_mesh,
    compiler_params=pltpu.CompilerParams(use_tc_tiling_on_sc=False),
    scratch_types=[
        pltpu.VMEM_SHARED((N, H), F32),
        pltpu.VMEM((RC, H), F32),
        pltpu.VMEM((CH, H), F32),
        pltpu.VMEM((CH,), jnp.int32),
        pltpu.VMEM((CH,), jnp.int32),
    ],
)(_spmm_body)


def _gather_body(idxs, emb, rs_a, rs_b, emb_g, rs_g, ibuf, gbuf64, gbuf32):
    c = lax.axis_index("c")
    s = lax.axis_index("s")

    def rwork(ci, rs_ref):
        def rstep(j, carry):
            ch = s * (GCH // NS) + j
            pltpu.sync_copy(idxs.at[pl.ds(ch * CH, CH)], ibuf)
            pltpu.sync_copy(rs_ref.at[ibuf], gbuf32)
            pltpu.sync_copy(gbuf32, rs_g.at[ci, pl.ds(ch * CH, CH)])
            return carry

        lax.fori_loop(0, GCH // NS, rstep, 0)

    def ework(ci):
        def estep(j, carry):
            ch = ci * (GCH // 2) + s * (GCH // NC // NS) + j
            pltpu.sync_copy(idxs.at[pl.ds(ch * CH, CH)], ibuf)
            pltpu.sync_copy(emb.at[ibuf], gbuf64)
            pltpu.sync_copy(gbuf64, emb_g.at[pl.ds(ch * CH, CH)])
            return carry

        lax.fori_loop(0, GCH // NC // NS, estep, 0)

    @pl.when(c == 0)
    def _():
        rwork(0, rs_a)
        ework(0)

    @pl.when(c == 1)
    def _():
        rwork(1, rs_b)
        ework(1)


_gather_kernel = functools.partial(
    pl.kernel,
    out_type=(
        jax.ShapeDtypeStruct((G, D), F32),
        jax.ShapeDtypeStruct((NC, G, H), F32),
    ),
    mesh=_mesh,
    compiler_params=pltpu.CompilerParams(use_tc_tiling_on_sc=False),
    scratch_types=[
        pltpu.VMEM((CH,), jnp.int32),
        pltpu.VMEM((CH, D), F32),
        pltpu.VMEM((CH, H), F32),
    ],
)(_gather_body)


_RB = 5000


def _prep_body(deg_ref, emb_ref, dr_ref, drc_ref, u1a_ref, u1b_ref):
    dr = lax.rsqrt(jnp.maximum(deg_ref[0, :, 0:1], 1.0))
    dc = lax.rsqrt(jnp.maximum(deg_ref[1, :, 0:1], 1.0))
    dr_ref[...] = dr
    drc_ref[...] = dr * dc
    u1 = dc * emb_ref[...]
    u1a_ref[...] = u1[:, :H]
    u1b_ref[...] = u1[:, H:]


def _prep(deg, emb):
    return pl.pallas_call(
        _prep_body,
        grid=(N // _RB,),
        in_specs=[
            pl.BlockSpec((NC, _RB, 16), lambda i: (0, i, 0)),
            pl.BlockSpec((_RB, D), lambda i: (i, 0)),
        ],
        out_specs=[
            pl.BlockSpec((_RB, 1), lambda i: (i, 0)),
            pl.BlockSpec((_RB, 1), lambda i: (i, 0)),
            pl.BlockSpec((_RB, H), lambda i: (i, 0)),
            pl.BlockSpec((_RB, H), lambda i: (i, 0)),
        ],
        out_shape=[
            jax.ShapeDtypeStruct((N, 1), F32),
            jax.ShapeDtypeStruct((N, 1), F32),
            jax.ShapeDtypeStruct((N, H), F32),
            jax.ShapeDtypeStruct((N, H), F32),
        ],
    )(deg, emb)


def _scale_body(s_ref, drc_ref, ua_ref, ub_ref):
    drc = drc_ref[...]
    ua_ref[...] = drc * s_ref[0]
    ub_ref[...] = drc * s_ref[1]


def _scale(s, drc):
    return pl.pallas_call(
        _scale_body,
        grid=(N // _RB,),
        in_specs=[
            pl.BlockSpec((NC, _RB, H), lambda i: (0, i, 0)),
            pl.BlockSpec((_RB, 1), lambda i: (i, 0)),
        ],
        out_specs=[
            pl.BlockSpec((_RB, H), lambda i: (i, 0)),
            pl.BlockSpec((_RB, H), lambda i: (i, 0)),
        ],
        out_shape=[
            jax.ShapeDtypeStruct((N, H), F32),
            jax.ShapeDtypeStruct((N, H), F32),
        ],
    )(s, drc)


def _fscale_body(s1_ref, s2_ref, s3_ref, dr_ref, ra_ref, rb_ref):
    dr = dr_ref[...]
    ra_ref[...] = dr * (s1_ref[0] + s2_ref[0] + s3_ref[0])
    rb_ref[...] = dr * (s1_ref[1] + s2_ref[1] + s3_ref[1])


def _fscale(s1, s2, s3, dr):
    sspec = pl.BlockSpec((NC, _RB, H), lambda i: (0, i, 0))
    return pl.pallas_call(
        _fscale_body,
        grid=(N // _RB,),
        in_specs=[sspec, sspec, sspec,
                  pl.BlockSpec((_RB, 1), lambda i: (i, 0))],
        out_specs=[
            pl.BlockSpec((_RB, H), lambda i: (i, 0)),
            pl.BlockSpec((_RB, H), lambda i: (i, 0)),
        ],
        out_shape=[
            jax.ShapeDtypeStruct((N, H), F32),
            jax.ShapeDtypeStruct((N, H), F32),
        ],
    )(s1, s2, s3, dr)


def _final_body(eg_ref, rs_ref, u_ref, p_ref, n_ref, l2_ref):
    eg = eg_ref[...]
    fr = 0.25 * (eg + jnp.concatenate([rs_ref[0], rs_ref[1]], axis=1))
    u_ref[...] = fr[:B]
    p_ref[...] = fr[B:2 * B]
    n_ref[...] = fr[2 * B:]
    l2_ref[...] = jnp.sum(
        eg[:B] ** 2 + eg[B:2 * B] ** 2 + eg[2 * B:] ** 2,
        axis=1, keepdims=True)


def _final(emb_g, rs_g):
    return pl.pallas_call(
        _final_body,
        out_shape=[
            jax.ShapeDtypeStruct((B, D), F32),
            jax.ShapeDtypeStruct((B, D), F32),
            jax.ShapeDtypeStruct((B, D), F32),
            jax.ShapeDtypeStruct((B, 1), F32),
        ],
    )(emb_g, rs_g)


def kernel(embedding, edge_index, users, pos_items, neg_items):
    row_e = edge_index[0]
    col_e = edge_index[1]
    zeros16 = jnp.zeros((N, 16), F32)
    zeros32 = jnp.zeros((N, H), F32)
    ones16 = jnp.ones((CH, 16), F32)
    idx_all = jnp.concatenate(
        [users, N_USERS + pos_items, N_USERS + neg_items])

    deg = _deg_kernel(row_e, col_e, zeros16, ones16)
    dr, drc, u1a, u1b = _prep(deg, embedding)
    s1 = _spmm_kernel(row_e, col_e, u1a, u1b, zeros32)
    u2a, u2b = _scale(s1, drc)
    s2 = _spmm_kernel(row_e, col_e, u2a, u2b, zeros32)
    u3a, u3b = _scale(s2, drc)
    s3 = _spmm_kernel(row_e, col_e, u3a, u3b, zeros32)
    rs_a, rs_b = _fscale(s1, s2, s3, dr)
    emb_g, rs_g = _gather_kernel(idx_all, embedding, rs_a, rs_b)
    users_r, pos_r, neg_r, l2 = _final(emb_g, rs_g)
    return users_r, pos_r, neg_r, l2.reshape(B)


# speedup vs baseline: 14.2911x; 1.9823x over previous
import functools

import jax
import jax.numpy as jnp
from jax import lax
from jax.experimental import pallas as pl
from jax.experimental.pallas import tpu as pltpu
from jax.experimental.pallas import tpu_sc as plsc

N_USERS = 25000
N = 50000
E = 800000
D = 64
H = 32
B = 4096
G = 3 * B
CH = 128
NCH = E // CH
NC = 2
NS = 16
CPT = 392
ECHP = NS * CPT
NA = N + 48
GB = 4
NPAIR = CPT // (2 * GB)
RC = 250
NRC = N // RC
RCPT = 13
GCH = G // CH
F32 = jnp.float32

_mesh = plsc.VectorSubcoreMesh(
    core_axis_name="c", subcore_axis_name="s", num_cores=NC, num_subcores=NS)


def _deg_body(row_e, col_e, zeros16, ones16, deg_out, acc, stg, ones_v, ibuf):
    c = lax.axis_index("c")
    s = lax.axis_index("s")
    pltpu.sync_copy(ones16, ones_v)

    def zero(j, carry):
        k = s + NS * j

        @pl.when(k < NRC)
        def _():
            pltpu.sync_copy(zeros16.at[pl.ds(k * RC, RC)], stg)
            pltpu.sync_copy(stg, acc.at[pl.ds(k * RC, RC)])

        return carry

    lax.fori_loop(0, RCPT, zero, 0)
    plsc.subcore_barrier()

    def work(e_ref):
        def step(j, carry):
            ch = s * CPT + j

            @pl.when(ch < NCH)
            def _():
                pltpu.sync_copy(e_ref.at[ch], ibuf)
                pltpu.sync_copy(ones_v, acc.at[ibuf], add=True)

            return carry

        lax.fori_loop(0, CPT, step, 0)

    @pl.when(c == 0)
    def _():
        work(row_e)

    @pl.when(c == 1)
    def _():
        work(col_e)

    plsc.subcore_barrier()

    def copy_out(ci):
        def cstep(j, carry):
            k = s + NS * j

            @pl.when(k < NRC)
            def _():
                pltpu.sync_copy(acc.at[pl.ds(k * RC, RC)], stg)
                pltpu.sync_copy(stg, deg_out.at[ci, pl.ds(k * RC, RC)])

            return carry

        lax.fori_loop(0, RCPT, cstep, 0)

    @pl.when(c == 0)
    def _():
        copy_out(0)

    @pl.when(c == 1)
    def _():
        copy_out(1)


_deg_kernel = functools.partial(
    pl.kernel,
    out_type=jax.ShapeDtypeStruct((NC, N, 16), F32),
    mesh=_mesh,
    compiler_params=pltpu.CompilerParams(use_tc_tiling_on_sc=False),
    scratch_types=[
        pltpu.VMEM_SHARED((N, 16), F32),
        pltpu.VMEM((RC, 16), F32),
        pltpu.VMEM((CH, 16), F32),
        pltpu.VMEM((CH,), jnp.int32),
    ],
)(_deg_body)


def _spmm_body(row_e, col_e, u_a, u_b, zeros32, s_out,
               acc, stg, g0, g1, g2, g3, cba, rba, cbb, rbb, gsem, ssem):
    c = lax.axis_index("c")
    s = lax.axis_index("s")
    gbufs = (g0, g1, g2, g3)

    def zero(j, carry):
        k = s + NS * j

        @pl.when(k < NRC)
        def _():
            pltpu.sync_copy(zeros32.at[pl.ds(k * RC, RC)], stg)
            pltpu.sync_copy(stg, acc.at[pl.ds(k * RC, RC)])

        return carry

    lax.fori_loop(0, RCPT, zero, 0)
    plsc.subcore_barrier()

    c0 = s * CPT

    def work(u_ref):
        def load_idx(base, cb, rb):
            pltpu.sync_copy(col_e.at[pl.ds(base, GB)], cb)
            pltpu.sync_copy(row_e.at[pl.ds(base, GB)], rb)

        def issue_gathers(cb):
            for b in range(GB):
                pltpu.async_copy(u_ref.at[cb.at[b]], gbufs[b], gsem.at[b])

        def scatter_group(rb):
            for b in range(GB):
                pltpu.make_async_copy(
                    u_ref.at[cb_dummy.at[b]], gbufs[b], gsem.at[b]).wait()
                pltpu.async_copy(
                    gbufs[b], acc.at[rb.at[b]], ssem.at[b], add=True)

        def wait_scatters(rb):
            for b in range(GB):
                pltpu.make_async_copy(
                    gbufs[b], acc.at[rb.at[b]], ssem.at[b]).wait()

        cb_dummy = cba

        load_idx(c0, cba, rba)
        issue_gathers(cba)

        def pair(p, carry):
            base_b = c0 + (2 * p + 1) * GB
            load_idx(base_b, cbb, rbb)
            scatter_group(rba)
            for b in range(GB):
                pltpu.make_async_copy(
                    gbufs[b], acc.at[rba.at[b]], ssem.at[b]).wait()
                pltpu.async_copy(u_ref.at[cbb.at[b]], gbufs[b], gsem.at[b])

            @pl.when(p < NPAIR - 1)
            def _():
                load_idx(c0 + (2 * p + 2) * GB, cba, rba)

            scatter_group(rbb)

            @pl.when(p < NPAIR - 1)
            def _():
                for b in range(GB):
                    pltpu.make_async_copy(
                        gbufs[b], acc.at[rbb.at[b]], ssem.at[b]).wait()
                    pltpu.async_copy(u_ref.at[cba.at[b]], gbufs[b], gsem.at[b])

            return carry

        lax.fori_loop(0, NPAIR, pair, 0)
        wait_scatters(rbb)

    @pl.when(c == 0)
    def _():
        work(u_a)

    @pl.when(c == 1)
    def _():
        work(u_b)

    plsc.subcore_barrier()

    def copy_out(ci):
        def cstep(j, carry):
            k = s + NS * j

            @pl.when(k < NRC)
            def _():
                pltpu.sync_copy(acc.at[pl.ds(k * RC, RC)], stg)
                pltpu.sync_copy(stg, s_out.at[ci, pl.ds(k * RC, RC)])

            return carry

        lax.fori_loop(0, RCPT, cstep, 0)

    @pl.when(c == 0)
    def _():
        copy_out(0)

    @pl.when(c == 1)
    def _():
        copy_out(1)


_spmm_kernel = functools.partial(
    pl.kernel,
    out_type=jax.ShapeDtypeStruct((NC, N, H), F32),
    mesh=_mesh,
    compiler_params=pltpu.CompilerParams(use_tc_tiling_on_sc=False),
    scratch_types=[
        pltpu.VMEM_SHARED((NA, H), F32),
        pltpu.VMEM((RC, H), F32),
        pltpu.VMEM((CH, H), F32),
        pltpu.VMEM((CH, H), F32),
        pltpu.VMEM((CH, H), F32),
        pltpu.VMEM((CH, H), F32),
        pltpu.VMEM((GB, CH), jnp.int32),
        pltpu.VMEM((GB, CH), jnp.int32),
        pltpu.VMEM((GB, CH), jnp.int32),
        pltpu.VMEM((GB, CH), jnp.int32),
        pltpu.SemaphoreType.DMA((GB,)),
        pltpu.SemaphoreType.DMA((GB,)),
    ],
)(_spmm_body)


def _gather_body(idxs, emb, rs_a, rs_b, emb_g, rs_g, ibuf, gbuf64, gbuf32):
    c = lax.axis_index("c")
    s = lax.axis_index("s")

    def rwork(ci, rs_ref):
        def rstep(j, carry):
            ch = s * (GCH // NS) + j
            pltpu.sync_copy(idxs.at[pl.ds(ch * CH, CH)], ibuf)
            pltpu.sync_copy(rs_ref.at[ibuf], gbuf32)
            pltpu.sync_copy(gbuf32, rs_g.at[ci, pl.ds(ch * CH, CH)])
            return carry

        lax.fori_loop(0, GCH // NS, rstep, 0)

    def ework(ci):
        def estep(j, carry):
            ch = ci * (GCH // 2) + s * (GCH // NC // NS) + j
            pltpu.sync_copy(idxs.at[pl.ds(ch * CH, CH)], ibuf)
            pltpu.sync_copy(emb.at[ibuf], gbuf64)
            pltpu.sync_copy(gbuf64, emb_g.at[pl.ds(ch * CH, CH)])
            return carry

        lax.fori_loop(0, GCH // NC // NS, estep, 0)

    @pl.when(c == 0)
    def _():
        rwork(0, rs_a)
        ework(0)

    @pl.when(c == 1)
    def _():
        rwork(1, rs_b)
        ework(1)


_gather_kernel = functools.partial(
    pl.kernel,
    out_type=(
        jax.ShapeDtypeStruct((G, D), F32),
        jax.ShapeDtypeStruct((NC, G, H), F32),
    ),
    mesh=_mesh,
    compiler_params=pltpu.CompilerParams(use_tc_tiling_on_sc=False),
    scratch_types=[
        pltpu.VMEM((CH,), jnp.int32),
        pltpu.VMEM((CH, D), F32),
        pltpu.VMEM((CH, H), F32),
    ],
)(_gather_body)


_RB = 5000


def _prep_body(deg_ref, emb_ref, dr_ref, drc_ref, u1a_ref, u1b_ref):
    dr = lax.rsqrt(jnp.maximum(deg_ref[0, :, 0:1], 1.0))
    dc = lax.rsqrt(jnp.maximum(deg_ref[1, :, 0:1], 1.0))
    dr_ref[...] = dr
    drc_ref[...] = dr * dc
    u1 = dc * emb_ref[...]
    u1a_ref[...] = u1[:, :H]
    u1b_ref[...] = u1[:, H:]


def _prep(deg, emb):
    return pl.pallas_call(
        _prep_body,
        grid=(N // _RB,),
        in_specs=[
            pl.BlockSpec((NC, _RB, 16), lambda i: (0, i, 0)),
            pl.BlockSpec((_RB, D), lambda i: (i, 0)),
        ],
        out_specs=[
            pl.BlockSpec((_RB, 1), lambda i: (i, 0)),
            pl.BlockSpec((_RB, 1), lambda i: (i, 0)),
            pl.BlockSpec((_RB, H), lambda i: (i, 0)),
            pl.BlockSpec((_RB, H), lambda i: (i, 0)),
        ],
        out_shape=[
            jax.ShapeDtypeStruct((N, 1), F32),
            jax.ShapeDtypeStruct((N, 1), F32),
            jax.ShapeDtypeStruct((N, H), F32),
            jax.ShapeDtypeStruct((N, H), F32),
        ],
    )(deg, emb)


def _scale_body(s_ref, drc_ref, ua_ref, ub_ref):
    drc = drc_ref[...]
    ua_ref[...] = drc * s_ref[0]
    ub_ref[...] = drc * s_ref[1]


def _scale(s, drc):
    return pl.pallas_call(
        _scale_body,
        grid=(N // _RB,),
        in_specs=[
            pl.BlockSpec((NC, _RB, H), lambda i: (0, i, 0)),
            pl.BlockSpec((_RB, 1), lambda i: (i, 0)),
        ],
        out_specs=[
            pl.BlockSpec((_RB, H), lambda i: (i, 0)),
            pl.BlockSpec((_RB, H), lambda i: (i, 0)),
        ],
        out_shape=[
            jax.ShapeDtypeStruct((N, H), F32),
            jax.ShapeDtypeStruct((N, H), F32),
        ],
    )(s, drc)


def _fscale_body(s1_ref, s2_ref, s3_ref, dr_ref, ra_ref, rb_ref):
    dr = dr_ref[...]
    ra_ref[...] = dr * (s1_ref[0] + s2_ref[0] + s3_ref[0])
    rb_ref[...] = dr * (s1_ref[1] + s2_ref[1] + s3_ref[1])


def _fscale(s1, s2, s3, dr):
    sspec = pl.BlockSpec((NC, _RB, H), lambda i: (0, i, 0))
    return pl.pallas_call(
        _fscale_body,
        grid=(N // _RB,),
        in_specs=[sspec, sspec, sspec,
                  pl.BlockSpec((_RB, 1), lambda i: (i, 0))],
        out_specs=[
            pl.BlockSpec((_RB, H), lambda i: (i, 0)),
            pl.BlockSpec((_RB, H), lambda i: (i, 0)),
        ],
        out_shape=[
            jax.ShapeDtypeStruct((N, H), F32),
            jax.ShapeDtypeStruct((N, H), F32),
        ],
    )(s1, s2, s3, dr)


def _final_body(eg_ref, rs_ref, u_ref, p_ref, n_ref, l2_ref):
    eg = eg_ref[...]
    fr = 0.25 * (eg + jnp.concatenate([rs_ref[0], rs_ref[1]], axis=1))
    u_ref[...] = fr[:B]
    p_ref[...] = fr[B:2 * B]
    n_ref[...] = fr[2 * B:]
    l2_ref[...] = jnp.sum(
        eg[:B] ** 2 + eg[B:2 * B] ** 2 + eg[2 * B:] ** 2,
        axis=1, keepdims=True)


def _final(emb_g, rs_g):
    return pl.pallas_call(
        _final_body,
        out_shape=[
            jax.ShapeDtypeStruct((B, D), F32),
            jax.ShapeDtypeStruct((B, D), F32),
            jax.ShapeDtypeStruct((B, D), F32),
            jax.ShapeDtypeStruct((B, 1), F32),
        ],
    )(emb_g, rs_g)


def kernel(embedding, edge_index, users, pos_items, neg_items):
    npad = ECHP * CH - E
    row_e = jnp.concatenate(
        [edge_index[0], jnp.full((npad,), N, jnp.int32)]).reshape(ECHP, CH)
    col_e = jnp.concatenate(
        [edge_index[1], jnp.zeros((npad,), jnp.int32)]).reshape(ECHP, CH)
    zeros16 = jnp.zeros((N, 16), F32)
    zeros32 = jnp.zeros((N, H), F32)
    ones16 = jnp.ones((CH, 16), F32)
    idx_all = jnp.concatenate(
        [users, N_USERS + pos_items, N_USERS + neg_items])

    deg = _deg_kernel(row_e, col_e, zeros16, ones16)
    dr, drc, u1a, u1b = _prep(deg, embedding)
    s1 = _spmm_kernel(row_e, col_e, u1a, u1b, zeros32)
    u2a, u2b = _scale(s1, drc)
    s2 = _spmm_kernel(row_e, col_e, u2a, u2b, zeros32)
    u3a, u3b = _scale(s2, drc)
    s3 = _spmm_kernel(row_e, col_e, u3a, u3b, zeros32)
    rs_a, rs_b = _fscale(s1, s2, s3, dr)
    emb_g, rs_g = _gather_kernel(idx_all, embedding, rs_a, rs_b)
    users_r, pos_r, neg_r, l2 = _final(emb_g, rs_g)
    return users_r, pos_r, neg_r, l2.reshape(B)
